# baseline (speedup 1.0000x reference)
import functools

import jax
import jax.numpy as jnp
from jax import lax
from jax.experimental import pallas as pl
from jax.experimental.pallas import tpu as pltpu
from jax.experimental.pallas import tpu_sc as plsc

B_TOK = 16384 * 50
D = 64
NC, NS = 2, 16
NW = NC * NS
IDX_MINOR = 128
R = 4
CHUNK = R * IDX_MINOR
ROWS_PER_W = B_TOK // NW
IDX_ROWS = ROWS_PER_W // IDX_MINOR
N_CHUNKS = ROWS_PER_W // CHUNK
NBUF = 3

_mesh = plsc.VectorSubcoreMesh(
    core_axis_name="c", subcore_axis_name="s", num_cores=NC, num_subcores=NS
)


@functools.partial(
    pl.kernel,
    out_type=jax.ShapeDtypeStruct((B_TOK, D), jnp.float32),
    mesh=_mesh,
    compiler_params=pltpu.CompilerParams(use_tc_tiling_on_sc=False),
    scratch_types=[
        pltpu.VMEM((IDX_ROWS, IDX_MINOR), jnp.int32),
        pltpu.VMEM((NBUF, CHUNK, D), jnp.float32),
        pltpu.SemaphoreType.DMA((NBUF,)),
        pltpu.SemaphoreType.DMA((NBUF,)),
    ],
)
def _gather_kernel(x_hbm, table_hbm, out_hbm, idx_all, rows_v, sem_g, sem_s):
    wid = lax.axis_index("s") * NC + lax.axis_index("c")
    row0 = wid * IDX_ROWS

    def issue_gather(i, b):
        for j in range(R):
            pltpu.async_copy(
                table_hbm.at[idx_all.at[i * R + j]],
                rows_v.at[b, pl.ds(j * IDX_MINOR, IDX_MINOR)],
                sem_g.at[b],
            )

    def wait_gather(b):
        pltpu.make_async_copy(
            out_hbm.at[pl.ds(0, CHUNK)], rows_v.at[b], sem_g.at[b]
        ).wait()

    def out_slice(i):
        return out_hbm.at[pl.ds((row0 + i * R) * IDX_MINOR, CHUNK)]

    def issue_store(i, b):
        pltpu.async_copy(rows_v.at[b], out_slice(i), sem_s.at[b])

    def wait_store(i, b):
        pltpu.make_async_copy(rows_v.at[b], out_slice(i), sem_s.at[b]).wait()

    pltpu.sync_copy(x_hbm.at[pl.ds(row0, IDX_ROWS)], idx_all)

    issue_gather(0, 0)
    issue_gather(1, 1)
    wait_gather(0)
    issue_store(0, 0)
    issue_gather(2, 2)

    def body(i, carry):
        b = i % NBUF
        wait_gather(b)
        issue_store(i, b)
        bn = (i + 2) % NBUF
        wait_store(i - 1, bn)
        issue_gather(i + 2, bn)
        return carry

    lax.fori_loop(1, N_CHUNKS - 2, body, 0)

    wait_gather((N_CHUNKS - 2) % NBUF)
    issue_store(N_CHUNKS - 2, (N_CHUNKS - 2) % NBUF)
    wait_gather((N_CHUNKS - 1) % NBUF)
    issue_store(N_CHUNKS - 1, (N_CHUNKS - 1) % NBUF)
    for i in range(N_CHUNKS - 3, N_CHUNKS):
        wait_store(i, i % NBUF)


def kernel(x, table):
    x_flat = x.reshape(B_TOK // IDX_MINOR, IDX_MINOR)
    out = _gather_kernel(x_flat, table)
    return out.reshape(x.shape[0], x.shape[1], D)


# speedup vs baseline: 1.3750x; 1.3750x over previous
import functools

import jax
import jax.numpy as jnp
from jax import lax
from jax.experimental import pallas as pl
from jax.experimental.pallas import tpu as pltpu
from jax.experimental.pallas import tpu_sc as plsc

B_TOK = 16384 * 50
D = 64
NC, NS = 2, 16
NW = NC * NS
IDX_MINOR = 128
R = 4
CHUNK = R * IDX_MINOR
ROWS_PER_W = B_TOK // NW
IDX_ROWS = ROWS_PER_W // IDX_MINOR
N_CHUNKS = ROWS_PER_W // CHUNK
NBUF = 3

_mesh = plsc.VectorSubcoreMesh(
    core_axis_name="c", subcore_axis_name="s", num_cores=NC, num_subcores=NS
)


@functools.partial(
    pl.kernel,
    out_type=jax.ShapeDtypeStruct((B_TOK, D), jnp.float32),
    mesh=_mesh,
    compiler_params=pltpu.CompilerParams(use_tc_tiling_on_sc=False),
    scratch_types=[
        pltpu.VMEM((IDX_ROWS, IDX_MINOR), jnp.int32),
        pltpu.VMEM((NBUF, CHUNK, D), jnp.float32),
        pltpu.VMEM_SHARED((1000, D), jnp.float32),
        pltpu.SemaphoreType.DMA((NBUF,)),
        pltpu.SemaphoreType.DMA((NBUF,)),
    ],
)
def _gather_kernel(
    x_hbm, table_hbm, out_hbm, idx_all, rows_v, table_sh, sem_g, sem_s
):
    wid = lax.axis_index("s") * NC + lax.axis_index("c")
    row0 = wid * IDX_ROWS

    @pl.when(lax.axis_index("s") == 0)
    def _():
        pltpu.sync_copy(table_hbm, table_sh)

    plsc.subcore_barrier()

    def issue_gather(i, b):
        for j in range(R):
            pltpu.async_copy(
                table_sh.at[idx_all.at[i * R + j]],
                rows_v.at[b, pl.ds(j * IDX_MINOR, IDX_MINOR)],
                sem_g.at[b],
            )

    def wait_gather(b):
        pltpu.make_async_copy(
            out_hbm.at[pl.ds(0, CHUNK)], rows_v.at[b], sem_g.at[b]
        ).wait()

    def out_slice(i):
        return out_hbm.at[pl.ds((row0 + i * R) * IDX_MINOR, CHUNK)]

    def issue_store(i, b):
        pltpu.async_copy(rows_v.at[b], out_slice(i), sem_s.at[b])

    def wait_store(i, b):
        pltpu.make_async_copy(rows_v.at[b], out_slice(i), sem_s.at[b]).wait()

    pltpu.sync_copy(x_hbm.at[pl.ds(row0, IDX_ROWS)], idx_all)

    issue_gather(0, 0)
    issue_gather(1, 1)
    wait_gather(0)
    issue_store(0, 0)
    issue_gather(2, 2)

    def body(i, carry):
        b = i % NBUF
        wait_gather(b)
        issue_store(i, b)
        bn = (i + 2) % NBUF
        wait_store(i - 1, bn)
        issue_gather(i + 2, bn)
        return carry

    lax.fori_loop(1, N_CHUNKS - 2, body, 0)

    wait_gather((N_CHUNKS - 2) % NBUF)
    issue_store(N_CHUNKS - 2, (N_CHUNKS - 2) % NBUF)
    wait_gather((N_CHUNKS - 1) % NBUF)
    issue_store(N_CHUNKS - 1, (N_CHUNKS - 1) % NBUF)
    for i in range(N_CHUNKS - 3, N_CHUNKS):
        wait_store(i, i % NBUF)


def kernel(x, table):
    x_flat = x.reshape(B_TOK // IDX_MINOR, IDX_MINOR)
    out = _gather_kernel(x_flat, table)
    return out.reshape(x.shape[0], x.shape[1], D)
